# baseline (speedup 1.0000x reference)
import jax
import jax.numpy as jnp
from jax import lax
from jax.experimental import pallas as pl
from jax.experimental.pallas import tpu as pltpu
from jax.experimental.pallas import tpu_sc as plsc

N = 10000
E = 320000
D = 128

NC = 2
NS = 16
CHUNK = 128
Dh = D // NC
NCH = 160
NQ = NCH // 4
EP = NS * NCH * CHUNK
CEXT = NCH + 4
NP = 10240
STRIPE = NP // NS


def _sc_body(xcat_hbm, row_hbm, col_hbm,
             agg0_hbm, agg1_hbm, deg0_hbm, deg1_hbm,
             agg_s, deg_s,
             rbuf, cbuf, g0, g1, g2, g3, zbuf, ones_v,
             sg0, sg1, sg2, sg3, ss0, ss1, ss2, ss3, sdA, sdB):
    c = lax.axis_index("c")
    s = lax.axis_index("s")
    gb = [g0, g1, g2, g3]
    sg = [sg0, sg1, sg2, sg3]
    ss = [ss0, ss1, ss2, ss3]
    zeros16 = jnp.zeros((16,), jnp.float32)
    ones16 = jnp.ones((16,), jnp.float32)

    def zrow(r, _):
        def zcol(k, _):
            g0[r, pl.ds(k * 16, 16)] = zeros16
            return 0
        return lax.fori_loop(0, Dh // 16, zcol, 0)
    lax.fori_loop(0, CHUNK, zrow, 0)

    def z1(i, _):
        zbuf[pl.ds(i * 16, 16)] = zeros16
        ones_v[pl.ds(i * 16, 16)] = ones16
        return 0
    lax.fori_loop(0, CHUNK // 16, z1, 0)

    pltpu.sync_copy(row_hbm.at[s], rbuf)
    pltpu.sync_copy(col_hbm.at[s], cbuf)

    def zagg(k, _):
        pltpu.sync_copy(g0, agg_s.at[pl.ds(s * STRIPE + k * CHUNK, CHUNK)])
        pltpu.sync_copy(zbuf, deg_s.at[pl.ds(s * STRIPE + k * CHUNK, CHUNK)])
        return 0
    lax.fori_loop(0, STRIPE // CHUNK, zagg, 0)

    plsc.subcore_barrier()

    xh = xcat_hbm.at[c]

    for k in range(3):
        pltpu.async_copy(xh.at[cbuf.at[k]], gb[k], sg[k])

    def quad(q, _):
        for k in range(4):
            j = 4 * q + k
            pltpu.make_async_copy(xh.at[cbuf.at[j]], gb[k], sg[k]).wait()
            pltpu.async_copy(gb[k], agg_s.at[rbuf.at[j]], ss[k], add=True)

            dsem = sdA if k < 2 else sdB
            @pl.when(c == (k % 2))
            def _():
                @pl.when(q > 0)
                def _():
                    pltpu.make_async_copy(ones_v, deg_s.at[rbuf.at[j]],
                                          dsem).wait()
                pltpu.async_copy(ones_v, deg_s.at[rbuf.at[j]], dsem, add=True)

            kp = (k + 3) % 4
            if k == 0:
                @pl.when(q > 0)
                def _():
                    pltpu.make_async_copy(gb[kp], agg_s.at[rbuf.at[j]],
                                          ss[kp]).wait()
            else:
                pltpu.make_async_copy(gb[kp], agg_s.at[rbuf.at[j]],
                                      ss[kp]).wait()
            pltpu.async_copy(xh.at[cbuf.at[j + 3]], gb[kp], sg[kp])
        return 0
    lax.fori_loop(0, NQ, quad, 0)

    pltpu.make_async_copy(gb[3], agg_s.at[rbuf.at[0]], ss[3]).wait()
    for k in range(3):
        pltpu.make_async_copy(xh.at[cbuf.at[0]], gb[k], sg[k]).wait()
    pltpu.make_async_copy(ones_v, deg_s.at[rbuf.at[0]], sdA).wait()
    pltpu.make_async_copy(ones_v, deg_s.at[rbuf.at[0]], sdB).wait()

    plsc.subcore_barrier()

    @pl.when(c == 0)
    def _():
        pltpu.sync_copy(agg_s.at[pl.ds(s * STRIPE, STRIPE)],
                        agg0_hbm.at[pl.ds(s * STRIPE, STRIPE)])

    @pl.when(c == 1)
    def _():
        pltpu.sync_copy(agg_s.at[pl.ds(s * STRIPE, STRIPE)],
                        agg1_hbm.at[pl.ds(s * STRIPE, STRIPE)])

    @pl.when((c == 0) & (s == 0))
    def _():
        pltpu.sync_copy(deg_s.at[pl.ds(0, N)], deg0_hbm)

    @pl.when((c == 1) & (s == 0))
    def _():
        pltpu.sync_copy(deg_s.at[pl.ds(0, N)], deg1_hbm)


def _sc_aggregate(xcat, row_r, col_r):
    mesh = plsc.VectorSubcoreMesh(
        core_axis_name="c", subcore_axis_name="s", num_cores=NC, num_subcores=NS
    )
    f32 = jnp.float32
    sem = pltpu.SemaphoreType.DMA
    return pl.kernel(
        _sc_body,
        out_type=[
            jax.ShapeDtypeStruct((NP, Dh), f32),
            jax.ShapeDtypeStruct((NP, Dh), f32),
            jax.ShapeDtypeStruct((N,), f32),
            jax.ShapeDtypeStruct((N,), f32),
        ],
        mesh=mesh,
        compiler_params=pltpu.CompilerParams(use_tc_tiling_on_sc=False),
        scratch_types=[
            pltpu.VMEM_SHARED((NP, Dh), f32),
            pltpu.VMEM_SHARED((NP,), f32),
            pltpu.VMEM((NCH, CHUNK), jnp.int32),
            pltpu.VMEM((CEXT, CHUNK), jnp.int32),
            pltpu.VMEM((CHUNK, Dh), f32),
            pltpu.VMEM((CHUNK, Dh), f32),
            pltpu.VMEM((CHUNK, Dh), f32),
            pltpu.VMEM((CHUNK, Dh), f32),
            pltpu.VMEM((CHUNK,), f32),
            pltpu.VMEM((CHUNK,), f32),
            sem, sem, sem, sem,
            sem, sem, sem, sem,
            sem, sem,
        ],
    )(xcat, row_r, col_r)


def _tc_body(a0_ref, a1_ref, d0_ref, d1_ref, w0_ref, w1_ref, b_ref, o_ref):
    deg = d0_ref[...] + d1_ref[...]
    dinv = jnp.where(deg > 0.0, 1.0 / deg, 0.0)
    a0 = a0_ref[...] * dinv
    a1 = a1_ref[...] * dinv
    o_ref[...] = (
        jnp.dot(a0, w0_ref[...], preferred_element_type=jnp.float32)
        + jnp.dot(a1, w1_ref[...], preferred_element_type=jnp.float32)
        + b_ref[...]
    )


def _tc_epilogue(agg0, agg1, deg0, deg1, W0, W1, b2):
    Bn = 1000
    grid = (N // Bn,)
    return pl.pallas_call(
        _tc_body,
        grid=grid,
        in_specs=[
            pl.BlockSpec((Bn, Dh), lambda i: (i, 0)),
            pl.BlockSpec((Bn, Dh), lambda i: (i, 0)),
            pl.BlockSpec((Bn, 1), lambda i: (i, 0)),
            pl.BlockSpec((Bn, 1), lambda i: (i, 0)),
            pl.BlockSpec((Dh, D), lambda i: (0, 0)),
            pl.BlockSpec((Dh, D), lambda i: (0, 0)),
            pl.BlockSpec((1, D), lambda i: (0, 0)),
        ],
        out_specs=pl.BlockSpec((Bn, D), lambda i: (i, 0)),
        out_shape=jax.ShapeDtypeStruct((N, D), jnp.float32),
    )(agg0, agg1, deg0, deg1, W0, W1, b2)


def kernel(x, edge_index, W, b):
    row = edge_index[0].astype(jnp.int32)
    col = edge_index[1].astype(jnp.int32)
    pad = EP - E
    sink = N + (jnp.arange(pad, dtype=jnp.int32) % (NP - N))
    rowp = jnp.concatenate([row, sink])
    colp = jnp.concatenate([col, jnp.zeros((pad,), jnp.int32)])
    row_r = rowp.reshape(NS, NCH, CHUNK)
    col_r = jnp.concatenate(
        [colp.reshape(NS, NCH, CHUNK),
         jnp.zeros((NS, CEXT - NCH, CHUNK), jnp.int32)], axis=1)
    xp = jnp.concatenate([x, jnp.zeros((NP - N, D), x.dtype)], axis=0)
    xcat = jnp.stack([xp[:, :Dh], xp[:, Dh:]])

    agg0, agg1, deg0, deg1 = _sc_aggregate(xcat, row_r, col_r)

    W0 = W[:, :Dh].T
    W1 = W[:, Dh:].T
    return _tc_epilogue(agg0, agg1, deg0.reshape(N, 1), deg1.reshape(N, 1),
                        W0, W1, b.reshape(1, D))
